# Initial kernel scaffold; baseline (speedup 1.0000x reference)
import jax
import jax.numpy as jnp
from jax.experimental import pallas as pl


def kernel(x, direc, weight):
    raise NotImplementedError("write your pallas kernel here")


# speedup vs baseline: 1.5709x; 1.5709x over previous
import functools

import jax
import jax.numpy as jnp
from jax import lax
from jax.experimental import pallas as pl
from jax.experimental.pallas import tpu as pltpu
from jax.experimental.pallas import tpu_sc as plsc

_NW = 32
_LANES = 16


def _build_sc_gather(n_rows, d_model, boundary, chunk):
    per_w = n_rows // _NW
    n_chunks = per_w // chunk
    mesh = plsc.VectorSubcoreMesh(core_axis_name="c", subcore_axis_name="s")

    @functools.partial(
        pl.kernel,
        mesh=mesh,
        out_type=jax.ShapeDtypeStruct((n_rows, d_model), jnp.float32),
        scratch_types=[
            pltpu.VMEM((per_w,), jnp.int32),
            pltpu.VMEM((chunk, d_model), jnp.float32),
            pltpu.SemaphoreType.DMA,
        ],
    )
    def k(w_hbm, idx_hbm, out_hbm, idx_v, rows_v, sem):
        wid = lax.axis_index("s") * 2 + lax.axis_index("c")
        base = wid * per_w
        pltpu.sync_copy(idx_hbm.at[pl.ds(base, per_w)], idx_v)

        off = jnp.where(wid < boundary, 100, 200).astype(jnp.int32)

        def add_off(i, carry):
            sl = pl.ds(i * _LANES, _LANES)
            idx_v[sl] = idx_v[sl] + off
            return carry

        lax.fori_loop(0, per_w // _LANES, add_off, 0)

        def do_chunk(c, carry):
            idxs = idx_v.at[pl.ds(c * chunk, chunk)]
            pltpu.async_copy(w_hbm.at[idxs], rows_v, sem).wait()
            pltpu.sync_copy(rows_v, out_hbm.at[pl.ds(base + c * chunk, chunk)])
            return carry

        lax.fori_loop(0, n_chunks, do_chunk, 0)

    return k


def kernel(x, direc, weight):
    b, s = x.shape
    d = weight.shape[1]
    n = b * s
    if direc == "LR":
        boundary = _NW
    elif direc == "RL":
        boundary = 0
    else:
        boundary = _NW // 2

    xf = x.reshape(n).astype(jnp.int32)
    out = _build_sc_gather(n, d, boundary, chunk=40)(weight, xf)
    return out.reshape(b, s, d)


# speedup vs baseline: 1.5872x; 1.0104x over previous
import functools

import jax
import jax.numpy as jnp
from jax import lax
from jax.experimental import pallas as pl
from jax.experimental.pallas import tpu as pltpu
from jax.experimental.pallas import tpu_sc as plsc

_NW = 32
_LANES = 16


def _build_sc_gather(n_rows, d_model, boundary, chunk):
    per_w = n_rows // _NW
    n_chunks = per_w // chunk
    assert n_chunks % 2 == 0 and chunk % 8 == 0
    n_pairs = n_chunks // 2
    mesh = plsc.VectorSubcoreMesh(core_axis_name="c", subcore_axis_name="s")

    @functools.partial(
        pl.kernel,
        mesh=mesh,
        out_type=jax.ShapeDtypeStruct((n_rows, d_model), jnp.float32),
        scratch_types=[
            pltpu.VMEM((per_w,), jnp.int32),
            pltpu.VMEM((chunk, d_model), jnp.float32),
            pltpu.VMEM((chunk, d_model), jnp.float32),
            pltpu.SemaphoreType.DMA,
            pltpu.SemaphoreType.DMA,
            pltpu.SemaphoreType.DMA,
            pltpu.SemaphoreType.DMA,
        ],
    )
    def k(w_hbm, idx_hbm, out_hbm, idx_v, buf_a, buf_b, ga, gb, sa, sb):
        wid = lax.axis_index("s") * 2 + lax.axis_index("c")
        base = wid * per_w
        pltpu.sync_copy(idx_hbm.at[pl.ds(base, per_w)], idx_v)

        off = jnp.where(wid < boundary, 100, 200).astype(jnp.int32)

        def add_off(i, carry):
            sl = pl.ds(i * _LANES, _LANES)
            idx_v[sl] = idx_v[sl] + off
            return carry

        lax.fori_loop(0, per_w // _LANES, add_off, 0)

        def g_desc(c, buf, sem):
            idxs = idx_v.at[pl.ds(c * chunk, chunk)]
            return pltpu.make_async_copy(w_hbm.at[idxs], buf, sem)

        def s_desc(c, buf, sem):
            return pltpu.make_async_copy(
                buf, out_hbm.at[pl.ds(base + c * chunk, chunk)], sem
            )

        g_desc(0, buf_a, ga).start()

        def body(i, carry):
            c0 = 2 * i
            c1 = c0 + 1

            @pl.when(i >= 1)
            def _():
                s_desc(c1, buf_b, sb).wait()

            g_desc(c1, buf_b, gb).start()
            g_desc(c0, buf_a, ga).wait()
            s_desc(c0, buf_a, sa).start()
            s_desc(c0, buf_a, sa).wait()

            @pl.when(i + 1 < n_pairs)
            def _():
                g_desc(c0 + 2, buf_a, ga).start()

            g_desc(c1, buf_b, gb).wait()
            s_desc(c1, buf_b, sb).start()
            return carry

        lax.fori_loop(0, n_pairs, body, 0)
        s_desc(n_chunks - 1, buf_b, sb).wait()

    return k


def kernel(x, direc, weight):
    b, s = x.shape
    d = weight.shape[1]
    n = b * s
    if direc == "LR":
        boundary = _NW
    elif direc == "RL":
        boundary = 0
    else:
        boundary = _NW // 2

    xf = x.reshape(n).astype(jnp.int32)
    out = _build_sc_gather(n, d, boundary, chunk=40)(weight, xf)
    return out.reshape(b, s, d)


# speedup vs baseline: 2.1115x; 1.3303x over previous
import functools

import jax
import jax.numpy as jnp
from jax import lax
from jax.experimental import pallas as pl
from jax.experimental.pallas import tpu as pltpu
from jax.experimental.pallas import tpu_sc as plsc

_NW = 32
_LANES = 16


def _build_sc_gather(b, s, d_model, boundary, lo_off, hi_off):
    bat_per_w = b // _NW
    assert bat_per_w % 2 == 0
    n_pairs = bat_per_w // 2
    s_pad = (s + 7) // 8 * 8
    idx_per_w = bat_per_w * s_pad
    mesh = plsc.VectorSubcoreMesh(core_axis_name="c", subcore_axis_name="s")

    @functools.partial(
        pl.kernel,
        mesh=mesh,
        out_type=jax.ShapeDtypeStruct((b, s, d_model), jnp.float32),
        scratch_types=[
            pltpu.VMEM((idx_per_w,), jnp.int32),
            pltpu.VMEM((s, d_model), jnp.float32),
            pltpu.VMEM((s, d_model), jnp.float32),
            pltpu.SemaphoreType.DMA,
            pltpu.SemaphoreType.DMA,
            pltpu.SemaphoreType.DMA,
            pltpu.SemaphoreType.DMA,
        ],
    )
    def k(w_hbm, idx_hbm, out_hbm, idx_v, buf_a, buf_b, ga, gb, sa, sb):
        wid = lax.axis_index("s") * 2 + lax.axis_index("c")
        base_b = wid * bat_per_w
        pltpu.sync_copy(idx_hbm.at[pl.ds(wid * idx_per_w, idx_per_w)], idx_v)

        off = jnp.where(wid < boundary, lo_off, hi_off).astype(jnp.int32)

        def add_off(i, carry):
            sl = pl.ds(i * _LANES, _LANES)
            idx_v[sl] = idx_v[sl] + off
            return carry

        lax.fori_loop(0, idx_per_w // _LANES, add_off, 0)

        def g_desc(cb, buf, sem):
            idxs = idx_v.at[pl.ds(cb * s_pad, s)]
            return pltpu.make_async_copy(w_hbm.at[idxs], buf, sem)

        def s_desc(cb, buf, sem):
            return pltpu.make_async_copy(buf, out_hbm.at[base_b + cb], sem)

        g_desc(0, buf_a, ga).start()

        def body(i, carry):
            c0 = 2 * i
            c1 = c0 + 1

            @pl.when(i >= 1)
            def _():
                s_desc(c1, buf_b, sb).wait()

            g_desc(c1, buf_b, gb).start()
            g_desc(c0, buf_a, ga).wait()
            s_desc(c0, buf_a, sa).start()
            s_desc(c0, buf_a, sa).wait()

            @pl.when(i + 1 < n_pairs)
            def _():
                g_desc(c0 + 2, buf_a, ga).start()

            g_desc(c1, buf_b, gb).wait()
            s_desc(c1, buf_b, sb).start()
            return carry

        lax.fori_loop(0, n_pairs, body, 0)
        s_desc(bat_per_w - 1, buf_b, sb).wait()

    return k


def kernel(x, direc, weight):
    b, s = x.shape
    d = weight.shape[1]
    if direc == "LR":
        lo_off = hi_off = 100
    elif direc == "RL":
        lo_off = hi_off = 200
    else:
        lo_off, hi_off = 100, 200
    xi = x.astype(jnp.int32)
    s_pad = (s + 7) // 8 * 8
    if s_pad != s:
        xi = jnp.pad(xi, ((0, 0), (0, s_pad - s)))
    xi = xi.reshape(b * s_pad)
    return _build_sc_gather(b, s, d, _NW // 2, lo_off, hi_off)(weight, xi)
